# Initial kernel scaffold; baseline (speedup 1.0000x reference)
import jax
import jax.numpy as jnp
from jax.experimental import pallas as pl


def kernel(h_v, h_u, e_feat, edge_index, S, W1_a2u, b1_a2u, W2_a2u, b2_a2u, W1_u, b1_u, W2_u, b2_u, W1_u2a, b1_u2a, W2_u2a, b2_u2a, W1_a, b1_a, W2_a, b2_a):
    raise NotImplementedError("write your pallas kernel here")


# speedup vs baseline: 1.8106x; 1.8106x over previous
import functools

import jax
import jax.numpy as jnp
from jax import lax
from jax.experimental import pallas as pl
from jax.experimental.pallas import tpu as pltpu
from jax.experimental.pallas import tpu_sc as plsc

f32 = jnp.float32

NC = 2
NS = 16
CH = 80


def _edge_proj_body(x_ref, wp_ref, bp_ref, wq_ref, bq_ref, pe_ref, qe_ref):
    x = x_ref[...]
    p = jnp.dot(x, wp_ref[...], preferred_element_type=f32) + bp_ref[...]
    q = jnp.dot(x, wq_ref[...], preferred_element_type=f32) + bq_ref[...]
    pe_ref[0] = p[:, :128]
    pe_ref[1] = p[:, 128:]
    qe_ref[0] = q[:, :128]
    qe_ref[1] = q[:, 128:]


def _edge_proj(e_feat, wp, bp, wq, bq, block_e=4000):
    E = e_feat.shape[0]
    grid = (E // block_e,)
    return pl.pallas_call(
        _edge_proj_body,
        grid=grid,
        in_specs=[
            pl.BlockSpec((block_e, 128), lambda i: (i, 0)),
            pl.BlockSpec((128, 256), lambda i: (0, 0)),
            pl.BlockSpec((1, 256), lambda i: (0, 0)),
            pl.BlockSpec((128, 256), lambda i: (0, 0)),
            pl.BlockSpec((1, 256), lambda i: (0, 0)),
        ],
        out_specs=[
            pl.BlockSpec((2, block_e, 128), lambda i: (0, i, 0)),
            pl.BlockSpec((2, block_e, 128), lambda i: (0, i, 0)),
        ],
        out_shape=[
            jax.ShapeDtypeStruct((2, E, 128), f32),
            jax.ShapeDtypeStruct((2, E, 128), f32),
        ],
    )(e_feat, wp, bp, wq, bq)


def _node_proj_body(hv_ref, hu_ref, wpv_ref, wpu_ref, wqv_ref,
                    pv_ref, pu_ref, qv_ref):
    hv = hv_ref[...]
    hu = hu_ref[...]
    p = jnp.dot(hv, wpv_ref[...], preferred_element_type=f32)
    pv_ref[0] = p[:, :128]
    pv_ref[1] = p[:, 128:]
    p = jnp.dot(hu, wpu_ref[...], preferred_element_type=f32)
    pu_ref[0] = p[:, :128]
    pu_ref[1] = p[:, 128:]
    p = jnp.dot(hv, wqv_ref[...], preferred_element_type=f32)
    qv_ref[0] = p[:, :128]
    qv_ref[1] = p[:, 128:]


def _node_proj(h_v, h_u, wpv, wpu, wqv, block_n=2000):
    NV = h_v.shape[0]
    grid = (NV // block_n,)
    return pl.pallas_call(
        _node_proj_body,
        grid=grid,
        in_specs=[
            pl.BlockSpec((block_n, 128), lambda i: (i, 0)),
            pl.BlockSpec((block_n, 128), lambda i: (i, 0)),
            pl.BlockSpec((128, 256), lambda i: (0, 0)),
            pl.BlockSpec((128, 256), lambda i: (0, 0)),
            pl.BlockSpec((128, 256), lambda i: (0, 0)),
        ],
        out_specs=[
            pl.BlockSpec((2, block_n, 128), lambda i: (0, i, 0)),
            pl.BlockSpec((2, block_n, 128), lambda i: (0, i, 0)),
            pl.BlockSpec((2, block_n, 128), lambda i: (0, i, 0)),
        ],
        out_shape=[
            jax.ShapeDtypeStruct((2, NV, 128), f32),
            jax.ShapeDtypeStruct((2, NV, 128), f32),
            jax.ShapeDtypeStruct((2, NV, 128), f32),
        ],
    )(h_v, h_u, wpv, wpu, wqv)


def _make_edge_pass(Na, Nb, Nout, E):
    EPT = E // NS
    NCHUNK = EPT // CH
    Npad = -(-Nout // (NS * 8)) * (NS * 8)
    RO = Npad // NS

    out_type = [jax.ShapeDtypeStruct((2, Npad, 128), f32)]

    scratch = [
        pltpu.VMEM((CH,), jnp.int32),
        pltpu.VMEM((CH,), jnp.int32),
        pltpu.VMEM((CH,), jnp.int32),
        pltpu.VMEM((CH,), jnp.int32),
        pltpu.VMEM((CH, 128), f32),
        pltpu.VMEM((CH, 128), f32),
        pltpu.VMEM((CH, 128), f32),
        pltpu.VMEM_SHARED((Npad, 128), f32),
        pltpu.SemaphoreType.DMA,
        pltpu.SemaphoreType.DMA,
    ]

    mesh = plsc.VectorSubcoreMesh(core_axis_name="c", subcore_axis_name="s",
                                  num_cores=NC, num_subcores=NS)

    def body(taba, idxa_h, tabb, idxb_h, etab, zeros_h,
             acc_out, idxa, idxb, idxa_adj, idxb_adj,
             bufa, bufb, bufe, acc, sema, semb):
        c = lax.axis_index("c")
        s = lax.axis_index("s")

        pltpu.sync_copy(zeros_h.at[pl.ds(s * RO, RO)],
                        acc.at[pl.ds(s * RO, RO)])
        plsc.subcore_barrier()

        base0 = s * EPT

        def chunk(i, carry):
            base = base0 + i * CH
            pltpu.sync_copy(idxa_h.at[pl.ds(base, CH)], idxa)
            pltpu.sync_copy(idxb_h.at[pl.ds(base, CH)], idxb)
            for k in range(CH // 16):
                sl = pl.ds(k * 16, 16)
                idxa_adj[sl] = idxa[sl] + c * Na
                idxb_adj[sl] = idxb[sl] + c * Nb
            cpa = pltpu.async_copy(taba.at[idxa_adj], bufa, sema)
            cpb = pltpu.async_copy(tabb.at[idxb_adj], bufb, semb)
            pltpu.sync_copy(etab.at[pl.ds(c * E + base, CH)], bufe)
            cpa.wait()
            cpb.wait()

            def row(r, rcarry):
                for k in range(8):
                    sl = pl.ds(k * 16, 16)
                    v = bufa[r, sl] + bufb[r, sl] + bufe[r, sl]
                    bufa[r, sl] = jnp.maximum(v, 0.0)
                return rcarry
            lax.fori_loop(0, CH, row, 0)

            pltpu.sync_copy(bufa, acc.at[idxb], add=True)
            return carry

        lax.fori_loop(0, NCHUNK, chunk, 0)
        plsc.subcore_barrier()

        pltpu.sync_copy(acc.at[pl.ds(s * RO, RO)],
                        acc_out.at[c, pl.ds(s * RO, RO)])

    return functools.partial(
        pl.kernel, body, out_type=out_type, mesh=mesh,
        scratch_types=scratch)()


def _make_count_kernel(Ncnt, E):
    EPT = E // NS
    NCHUNK = EPT // CH
    Npad = -(-Ncnt // (NS * 8)) * (NS * 8)
    RO = Npad // NS

    out_type = [
        jax.ShapeDtypeStruct((Npad, 16), f32),
        jax.ShapeDtypeStruct((Npad, 16), f32),
    ]
    scratch = [
        pltpu.VMEM((CH,), jnp.int32),
        pltpu.VMEM((CH,), jnp.int32),
        pltpu.VMEM((CH, 16), f32),
        pltpu.VMEM((CH, 16), f32),
        pltpu.VMEM_SHARED((Npad, 16), f32),
        pltpu.VMEM_SHARED((Npad, 16), f32),
    ]
    mesh = plsc.VectorSubcoreMesh(core_axis_name="c", subcore_axis_name="s",
                                  num_cores=NC, num_subcores=NS)

    def body(idxa_h, idxb_h, cnta_out, cntb_out,
             idxa, idxb, ones_buf, zbuf, accda, accdb):
        c = lax.axis_index("c")
        s = lax.axis_index("s")

        def fill(i, carry):
            ones_buf[i] = jnp.ones((16,), f32)
            zbuf[i] = jnp.zeros((16,), f32)
            return carry
        lax.fori_loop(0, CH, fill, 0)
        for j in range(RO // CH):
            sl = pl.ds(s * RO + j * CH, CH)
            pltpu.sync_copy(zbuf, accda.at[sl])
            pltpu.sync_copy(zbuf, accdb.at[sl])
        plsc.subcore_barrier()

        base0 = s * EPT

        def chunk(i, carry):
            base = base0 + i * CH
            pltpu.sync_copy(idxa_h.at[pl.ds(base, CH)], idxa)
            pltpu.sync_copy(idxb_h.at[pl.ds(base, CH)], idxb)
            pltpu.sync_copy(ones_buf, accda.at[idxa], add=True)
            pltpu.sync_copy(ones_buf, accdb.at[idxb], add=True)
            return carry
        lax.fori_loop(0, NCHUNK, chunk, 0)
        plsc.subcore_barrier()

        @pl.when(c == 0)
        def _write_counts():
            sl2 = pl.ds(s * RO, RO)
            pltpu.sync_copy(accda.at[sl2], cnta_out.at[sl2])
            pltpu.sync_copy(accdb.at[sl2], cntb_out.at[sl2])

    return functools.partial(
        pl.kernel, body, out_type=out_type, mesh=mesh,
        scratch_types=scratch)()


def _mid_body(a0_ref, a1_ref, cntu_ref, hu_ref, s_ref,
              w2a0_ref, w2a1_ref, b2a_ref, w1uh_ref, w1um_ref, b1u_ref,
              w2u_ref, b2u_ref, wqu_ref, huo_ref, qu_ref):
    degu = jnp.sum(cntu_ref[...], axis=1)[:, None] * (1.0 / 16.0)
    m = jnp.dot(a0_ref[...], w2a0_ref[...], preferred_element_type=f32)
    m = m + jnp.dot(a1_ref[...], w2a1_ref[...], preferred_element_type=f32)
    m = (m + degu * b2a_ref[...]) / s_ref[0, 0]
    h1 = jnp.dot(hu_ref[...], w1uh_ref[...], preferred_element_type=f32)
    h1 = h1 + jnp.dot(m, w1um_ref[...], preferred_element_type=f32)
    h1 = jnp.maximum(h1 + b1u_ref[...], 0.0)
    huo = jnp.dot(h1, w2u_ref[...], preferred_element_type=f32) + b2u_ref[...]
    huo_ref[...] = huo
    q = jnp.dot(huo, wqu_ref[...], preferred_element_type=f32)
    qu_ref[0] = q[:, :128]
    qu_ref[1] = q[:, 128:]


def _mid(a0, a1, cntu, h_u, S, w2a0, w2a1, b2a, w1uh, w1um, b1u,
         w2u, b2u, wqu, block_n=2000):
    NU = h_u.shape[0]
    grid = (NU // block_n,)
    wspec = lambda r, c: pl.BlockSpec((r, c), lambda i: (0, 0))
    return pl.pallas_call(
        _mid_body,
        grid=grid,
        in_specs=[
            pl.BlockSpec((block_n, 128), lambda i: (i, 0)),
            pl.BlockSpec((block_n, 128), lambda i: (i, 0)),
            pl.BlockSpec((block_n, NS), lambda i: (i, 0)),
            pl.BlockSpec((block_n, 128), lambda i: (i, 0)),
            wspec(1, 1),
            wspec(128, 128), wspec(128, 128), wspec(1, 128),
            wspec(128, 128), wspec(128, 128), wspec(1, 128),
            wspec(128, 128), wspec(1, 128),
            wspec(128, 256),
        ],
        out_specs=[
            pl.BlockSpec((block_n, 128), lambda i: (i, 0)),
            pl.BlockSpec((2, block_n, 128), lambda i: (0, i, 0)),
        ],
        out_shape=[
            jax.ShapeDtypeStruct((NU, 128), f32),
            jax.ShapeDtypeStruct((2, NU, 128), f32),
        ],
    )(a0, a1, cntu, h_u, S, w2a0, w2a1, b2a, w1uh, w1um, b1u, w2u, b2u, wqu)


def _final_body(a0_ref, a1_ref, cntv_ref, hv_ref,
                w2b0_ref, w2b1_ref, b2b_ref, w1ah_ref, w1am_ref, b1a_ref,
                w2a_ref, b2a_ref, hvo_ref):
    degv = jnp.sum(cntv_ref[...], axis=1)[:, None] * (1.0 / 16.0)
    m = jnp.dot(a0_ref[...], w2b0_ref[...], preferred_element_type=f32)
    m = m + jnp.dot(a1_ref[...], w2b1_ref[...], preferred_element_type=f32)
    m = (m + degv * b2b_ref[...]) / jnp.maximum(degv, 1.0)
    h1 = jnp.dot(hv_ref[...], w1ah_ref[...], preferred_element_type=f32)
    h1 = h1 + jnp.dot(m, w1am_ref[...], preferred_element_type=f32)
    h1 = jnp.maximum(h1 + b1a_ref[...], 0.0)
    hvo_ref[...] = (jnp.dot(h1, w2a_ref[...], preferred_element_type=f32)
                    + b2a_ref[...])


def _final(a0, a1, cntv, h_v, w2b0, w2b1, b2b, w1ah, w1am, b1a,
           w2a, b2a, block_n=2000):
    NV = h_v.shape[0]
    grid = (NV // block_n,)
    wspec = lambda r, c: pl.BlockSpec((r, c), lambda i: (0, 0))
    return pl.pallas_call(
        _final_body,
        grid=grid,
        in_specs=[
            pl.BlockSpec((block_n, 128), lambda i: (i, 0)),
            pl.BlockSpec((block_n, 128), lambda i: (i, 0)),
            pl.BlockSpec((block_n, NS), lambda i: (i, 0)),
            pl.BlockSpec((block_n, 128), lambda i: (i, 0)),
            wspec(128, 128), wspec(128, 128), wspec(1, 128),
            wspec(128, 128), wspec(128, 128), wspec(1, 128),
            wspec(128, 128), wspec(1, 128),
        ],
        out_specs=pl.BlockSpec((block_n, 128), lambda i: (i, 0)),
        out_shape=jax.ShapeDtypeStruct((NV, 128), f32),
    )(a0, a1, cntv, h_v, w2b0, w2b1, b2b, w1ah, w1am, b1a, w2a, b2a)


@jax.jit
def kernel(h_v, h_u, e_feat, edge_index, S,
           W1_a2u, b1_a2u, W2_a2u, b2_a2u,
           W1_u, b1_u, W2_u, b2_u,
           W1_u2a, b1_u2a, W2_u2a, b2_u2a,
           W1_a, b1_a, W2_a, b2_a):
    NV, D = h_v.shape
    NU = h_u.shape[0]
    E = e_feat.shape[0]

    idx_v = edge_index[0]
    idx_u = edge_index[1]

    pe_cat, qe_cat = _edge_proj(
        e_feat,
        W1_a2u[2 * D:], b1_a2u[None, :],
        W1_u2a[2 * D:], b1_u2a[None, :])
    pe_cat = pe_cat.reshape(2 * E, 128)
    qe_cat = qe_cat.reshape(2 * E, 128)

    pv_cat, pu_cat, qv_cat = _node_proj(
        h_v, h_u, W1_a2u[:D], W1_a2u[D:2 * D], W1_u2a[D:2 * D])
    pv_cat = pv_cat.reshape(2 * NV, 128)
    pu_cat = pu_cat.reshape(2 * NU, 128)
    qv_cat = qv_cat.reshape(2 * NV, 128)

    npad = -(-max(NU, NV) // (NS * 8)) * (NS * 8)
    zeros_rows = jnp.zeros((npad, 128), f32)

    cnt_a, cnt_b = _make_count_kernel(max(NU, NV), E)(idx_v, idx_u)
    cnt_u = cnt_b[:NU]
    cnt_v = cnt_a[:NV]

    (au_cat,) = _make_edge_pass(NV, NU, NU, E)(
        pv_cat, idx_v, pu_cat, idx_u, pe_cat, zeros_rows)

    h_u_out, qu_cat = _mid(
        au_cat[0, :NU], au_cat[1, :NU], cnt_u, h_u, S.reshape(1, 1),
        W2_a2u[:D], W2_a2u[D:], b2_a2u[None, :],
        W1_u[:D], W1_u[D:], b1_u[None, :],
        W2_u, b2_u[None, :],
        W1_u2a[:D])
    qu_cat = qu_cat.reshape(2 * NU, 128)

    (av_cat,) = _make_edge_pass(NU, NV, NV, E)(
        qu_cat, idx_u, qv_cat, idx_v, qe_cat, zeros_rows)

    h_v_out = _final(
        av_cat[0, :NV], av_cat[1, :NV], cnt_v, h_v,
        W2_u2a[:D], W2_u2a[D:], b2_u2a[None, :],
        W1_a[:D], W1_a[D:], b1_a[None, :],
        W2_a, b2_a[None, :])

    return (h_v_out, h_u_out)


# speedup vs baseline: 2.1723x; 1.1998x over previous
import functools

import jax
import jax.numpy as jnp
from jax import lax
from jax.experimental import pallas as pl
from jax.experimental.pallas import tpu as pltpu
from jax.experimental.pallas import tpu_sc as plsc

f32 = jnp.float32

NC = 2
NS = 16
CH = 40


def _edge_proj_body(x_ref, wp_ref, bp_ref, wq_ref, bq_ref, pe_ref, qe_ref):
    x = x_ref[...]
    p = jnp.dot(x, wp_ref[...], preferred_element_type=f32) + bp_ref[...]
    q = jnp.dot(x, wq_ref[...], preferred_element_type=f32) + bq_ref[...]
    pe_ref[0] = p[:, :128]
    pe_ref[1] = p[:, 128:]
    qe_ref[0] = q[:, :128]
    qe_ref[1] = q[:, 128:]


def _edge_proj(e_feat, wp, bp, wq, bq, block_e=4000):
    E = e_feat.shape[0]
    grid = (E // block_e,)
    return pl.pallas_call(
        _edge_proj_body,
        grid=grid,
        in_specs=[
            pl.BlockSpec((block_e, 128), lambda i: (i, 0)),
            pl.BlockSpec((128, 256), lambda i: (0, 0)),
            pl.BlockSpec((1, 256), lambda i: (0, 0)),
            pl.BlockSpec((128, 256), lambda i: (0, 0)),
            pl.BlockSpec((1, 256), lambda i: (0, 0)),
        ],
        out_specs=[
            pl.BlockSpec((2, block_e, 128), lambda i: (0, i, 0)),
            pl.BlockSpec((2, block_e, 128), lambda i: (0, i, 0)),
        ],
        out_shape=[
            jax.ShapeDtypeStruct((2, E, 128), f32),
            jax.ShapeDtypeStruct((2, E, 128), f32),
        ],
    )(e_feat, wp, bp, wq, bq)


def _node_proj_body(hv_ref, hu_ref, wpv_ref, wpu_ref, wqv_ref,
                    pv_ref, pu_ref, qv_ref):
    hv = hv_ref[...]
    hu = hu_ref[...]
    p = jnp.dot(hv, wpv_ref[...], preferred_element_type=f32)
    pv_ref[0] = p[:, :128]
    pv_ref[1] = p[:, 128:]
    p = jnp.dot(hu, wpu_ref[...], preferred_element_type=f32)
    pu_ref[0] = p[:, :128]
    pu_ref[1] = p[:, 128:]
    p = jnp.dot(hv, wqv_ref[...], preferred_element_type=f32)
    qv_ref[0] = p[:, :128]
    qv_ref[1] = p[:, 128:]


def _node_proj(h_v, h_u, wpv, wpu, wqv, block_n=2000):
    NV = h_v.shape[0]
    grid = (NV // block_n,)
    return pl.pallas_call(
        _node_proj_body,
        grid=grid,
        in_specs=[
            pl.BlockSpec((block_n, 128), lambda i: (i, 0)),
            pl.BlockSpec((block_n, 128), lambda i: (i, 0)),
            pl.BlockSpec((128, 256), lambda i: (0, 0)),
            pl.BlockSpec((128, 256), lambda i: (0, 0)),
            pl.BlockSpec((128, 256), lambda i: (0, 0)),
        ],
        out_specs=[
            pl.BlockSpec((2, block_n, 128), lambda i: (0, i, 0)),
            pl.BlockSpec((2, block_n, 128), lambda i: (0, i, 0)),
            pl.BlockSpec((2, block_n, 128), lambda i: (0, i, 0)),
        ],
        out_shape=[
            jax.ShapeDtypeStruct((2, NV, 128), f32),
            jax.ShapeDtypeStruct((2, NV, 128), f32),
            jax.ShapeDtypeStruct((2, NV, 128), f32),
        ],
    )(h_v, h_u, wpv, wpu, wqv)


def _make_edge_pass(Na, Nb, Nout, E):
    EPT = E // NS
    NCHUNK = EPT // CH
    Npad = -(-Nout // (NS * 8)) * (NS * 8)
    RO = Npad // NS

    out_type = [jax.ShapeDtypeStruct((2, Npad, 128), f32)]

    scratch = (
        [pltpu.VMEM((CH,), jnp.int32)] * 8
        + [pltpu.VMEM((CH, 128), f32)] * 6
        + [pltpu.VMEM_SHARED((Npad, 128), f32)]
        + [pltpu.SemaphoreType.DMA] * 6
    )

    mesh = plsc.VectorSubcoreMesh(core_axis_name="c", subcore_axis_name="s",
                                  num_cores=NC, num_subcores=NS)

    def body(taba, idxa_h, tabb, idxb_h, etab, zeros_h, acc_out,
             ia0, ib0, aa0, ab0, ia1, ib1, aa1, ab1,
             ba0, bb0, be0, ba1, bb1, be1,
             acc, sa0, sb0, se0, sa1, sb1, se1):
        c = lax.axis_index("c")
        s = lax.axis_index("s")
        sets = [
            (ia0, ib0, aa0, ab0, ba0, bb0, be0, sa0, sb0, se0),
            (ia1, ib1, aa1, ab1, ba1, bb1, be1, sa1, sb1, se1),
        ]

        pltpu.sync_copy(zeros_h.at[pl.ds(s * RO, RO)],
                        acc.at[pl.ds(s * RO, RO)])
        plsc.subcore_barrier()

        base0 = s * EPT

        def load_and_fire(i, bset):
            ia, ib, aa, ab, ba, bb, be, sa, sb, se = bset
            base = base0 + i * CH
            pltpu.sync_copy(idxa_h.at[pl.ds(base, CH)], ia)
            pltpu.sync_copy(idxb_h.at[pl.ds(base, CH)], ib)
            offs = list(range(0, CH - 15, 16))
            if CH % 16:
                offs.append(CH - 16)
            for off in offs:
                sl = pl.ds(off, 16)
                aa[sl] = ia[sl] + c * Na
                ab[sl] = ib[sl] + c * Nb
            pltpu.async_copy(taba.at[aa], ba, sa)
            pltpu.async_copy(tabb.at[ab], bb, sb)
            pltpu.async_copy(etab.at[pl.ds(c * E + base, CH)], be, se)

        def drain_compute_scatter(bset):
            ia, ib, aa, ab, ba, bb, be, sa, sb, se = bset
            pltpu.make_async_copy(taba.at[aa], ba, sa).wait()
            pltpu.make_async_copy(tabb.at[ab], bb, sb).wait()
            pltpu.make_async_copy(etab.at[pl.ds(0, CH)], be, se).wait()

            def row(r, rcarry):
                for k in range(8):
                    sl = pl.ds(k * 16, 16)
                    v = ba[r, sl] + bb[r, sl] + be[r, sl]
                    ba[r, sl] = jnp.maximum(v, 0.0)
                return rcarry
            lax.fori_loop(0, CH, row, 0)
            pltpu.sync_copy(ba, acc.at[ib], add=True)

        load_and_fire(0, sets[0])

        def pair(i2, carry):
            for b in (0, 1):
                i = i2 * 2 + b

                @pl.when(i + 1 < NCHUNK)
                def _prefetch():
                    load_and_fire(i + 1, sets[1 - b])
                drain_compute_scatter(sets[b])
            return carry

        lax.fori_loop(0, NCHUNK // 2, pair, 0)
        plsc.subcore_barrier()

        pltpu.sync_copy(acc.at[pl.ds(s * RO, RO)],
                        acc_out.at[c, pl.ds(s * RO, RO)])

    return functools.partial(
        pl.kernel, body, out_type=out_type, mesh=mesh,
        scratch_types=scratch)()


def _make_count_kernel(Ncnt, E):
    CC = 80
    EPT = E // NS
    NCHUNK = EPT // CC
    Npad = -(-Ncnt // (NS * 8)) * (NS * 8)
    RO = Npad // NS

    out_type = [
        jax.ShapeDtypeStruct((Npad, 16), f32),
        jax.ShapeDtypeStruct((Npad, 16), f32),
    ]
    scratch = [
        pltpu.VMEM((CC,), jnp.int32),
        pltpu.VMEM((CC,), jnp.int32),
        pltpu.VMEM((CC, 16), f32),
        pltpu.VMEM((CC, 16), f32),
        pltpu.VMEM_SHARED((Npad, 16), f32),
        pltpu.VMEM_SHARED((Npad, 16), f32),
    ]
    mesh = plsc.VectorSubcoreMesh(core_axis_name="c", subcore_axis_name="s",
                                  num_cores=NC, num_subcores=NS)

    def body(idxa_h, idxb_h, cnta_out, cntb_out,
             idxa, idxb, ones_buf, zbuf, accda, accdb):
        c = lax.axis_index("c")
        s = lax.axis_index("s")

        def fill(i, carry):
            ones_buf[i] = jnp.ones((16,), f32)
            zbuf[i] = jnp.zeros((16,), f32)
            return carry
        lax.fori_loop(0, CC, fill, 0)
        for j in range(RO // CC):
            sl = pl.ds(s * RO + j * CC, CC)
            pltpu.sync_copy(zbuf, accda.at[sl])
            pltpu.sync_copy(zbuf, accdb.at[sl])
        plsc.subcore_barrier()

        base0 = s * EPT

        def chunk(i, carry):
            base = base0 + i * CC
            pltpu.sync_copy(idxa_h.at[pl.ds(base, CC)], idxa)
            pltpu.sync_copy(idxb_h.at[pl.ds(base, CC)], idxb)
            pltpu.sync_copy(ones_buf, accda.at[idxa], add=True)
            pltpu.sync_copy(ones_buf, accdb.at[idxb], add=True)
            return carry
        lax.fori_loop(0, NCHUNK, chunk, 0)
        plsc.subcore_barrier()

        @pl.when(c == 0)
        def _write_counts():
            sl2 = pl.ds(s * RO, RO)
            pltpu.sync_copy(accda.at[sl2], cnta_out.at[sl2])
            pltpu.sync_copy(accdb.at[sl2], cntb_out.at[sl2])

    return functools.partial(
        pl.kernel, body, out_type=out_type, mesh=mesh,
        scratch_types=scratch)()


def _mid_body(a0_ref, a1_ref, cntu_ref, hu_ref, s_ref,
              w2a0_ref, w2a1_ref, b2a_ref, w1uh_ref, w1um_ref, b1u_ref,
              w2u_ref, b2u_ref, wqu_ref, huo_ref, qu_ref):
    degu = jnp.sum(cntu_ref[...], axis=1)[:, None] * (1.0 / 16.0)
    m = jnp.dot(a0_ref[...], w2a0_ref[...], preferred_element_type=f32)
    m = m + jnp.dot(a1_ref[...], w2a1_ref[...], preferred_element_type=f32)
    m = (m + degu * b2a_ref[...]) / s_ref[0, 0]
    h1 = jnp.dot(hu_ref[...], w1uh_ref[...], preferred_element_type=f32)
    h1 = h1 + jnp.dot(m, w1um_ref[...], preferred_element_type=f32)
    h1 = jnp.maximum(h1 + b1u_ref[...], 0.0)
    huo = jnp.dot(h1, w2u_ref[...], preferred_element_type=f32) + b2u_ref[...]
    huo_ref[...] = huo
    q = jnp.dot(huo, wqu_ref[...], preferred_element_type=f32)
    qu_ref[0] = q[:, :128]
    qu_ref[1] = q[:, 128:]


def _mid(a0, a1, cntu, h_u, S, w2a0, w2a1, b2a, w1uh, w1um, b1u,
         w2u, b2u, wqu, block_n=2000):
    NU = h_u.shape[0]
    grid = (NU // block_n,)
    wspec = lambda r, c: pl.BlockSpec((r, c), lambda i: (0, 0))
    return pl.pallas_call(
        _mid_body,
        grid=grid,
        in_specs=[
            pl.BlockSpec((block_n, 128), lambda i: (i, 0)),
            pl.BlockSpec((block_n, 128), lambda i: (i, 0)),
            pl.BlockSpec((block_n, NS), lambda i: (i, 0)),
            pl.BlockSpec((block_n, 128), lambda i: (i, 0)),
            wspec(1, 1),
            wspec(128, 128), wspec(128, 128), wspec(1, 128),
            wspec(128, 128), wspec(128, 128), wspec(1, 128),
            wspec(128, 128), wspec(1, 128),
            wspec(128, 256),
        ],
        out_specs=[
            pl.BlockSpec((block_n, 128), lambda i: (i, 0)),
            pl.BlockSpec((2, block_n, 128), lambda i: (0, i, 0)),
        ],
        out_shape=[
            jax.ShapeDtypeStruct((NU, 128), f32),
            jax.ShapeDtypeStruct((2, NU, 128), f32),
        ],
    )(a0, a1, cntu, h_u, S, w2a0, w2a1, b2a, w1uh, w1um, b1u, w2u, b2u, wqu)


def _final_body(a0_ref, a1_ref, cntv_ref, hv_ref,
                w2b0_ref, w2b1_ref, b2b_ref, w1ah_ref, w1am_ref, b1a_ref,
                w2a_ref, b2a_ref, hvo_ref):
    degv = jnp.sum(cntv_ref[...], axis=1)[:, None] * (1.0 / 16.0)
    m = jnp.dot(a0_ref[...], w2b0_ref[...], preferred_element_type=f32)
    m = m + jnp.dot(a1_ref[...], w2b1_ref[...], preferred_element_type=f32)
    m = (m + degv * b2b_ref[...]) / jnp.maximum(degv, 1.0)
    h1 = jnp.dot(hv_ref[...], w1ah_ref[...], preferred_element_type=f32)
    h1 = h1 + jnp.dot(m, w1am_ref[...], preferred_element_type=f32)
    h1 = jnp.maximum(h1 + b1a_ref[...], 0.0)
    hvo_ref[...] = (jnp.dot(h1, w2a_ref[...], preferred_element_type=f32)
                    + b2a_ref[...])


def _final(a0, a1, cntv, h_v, w2b0, w2b1, b2b, w1ah, w1am, b1a,
           w2a, b2a, block_n=2000):
    NV = h_v.shape[0]
    grid = (NV // block_n,)
    wspec = lambda r, c: pl.BlockSpec((r, c), lambda i: (0, 0))
    return pl.pallas_call(
        _final_body,
        grid=grid,
        in_specs=[
            pl.BlockSpec((block_n, 128), lambda i: (i, 0)),
            pl.BlockSpec((block_n, 128), lambda i: (i, 0)),
            pl.BlockSpec((block_n, NS), lambda i: (i, 0)),
            pl.BlockSpec((block_n, 128), lambda i: (i, 0)),
            wspec(128, 128), wspec(128, 128), wspec(1, 128),
            wspec(128, 128), wspec(128, 128), wspec(1, 128),
            wspec(128, 128), wspec(1, 128),
        ],
        out_specs=pl.BlockSpec((block_n, 128), lambda i: (i, 0)),
        out_shape=jax.ShapeDtypeStruct((NV, 128), f32),
    )(a0, a1, cntv, h_v, w2b0, w2b1, b2b, w1ah, w1am, b1a, w2a, b2a)


@jax.jit
def kernel(h_v, h_u, e_feat, edge_index, S,
           W1_a2u, b1_a2u, W2_a2u, b2_a2u,
           W1_u, b1_u, W2_u, b2_u,
           W1_u2a, b1_u2a, W2_u2a, b2_u2a,
           W1_a, b1_a, W2_a, b2_a):
    NV, D = h_v.shape
    NU = h_u.shape[0]
    E = e_feat.shape[0]

    idx_v = edge_index[0]
    idx_u = edge_index[1]

    pe_cat, qe_cat = _edge_proj(
        e_feat,
        W1_a2u[2 * D:], b1_a2u[None, :],
        W1_u2a[2 * D:], b1_u2a[None, :])
    pe_cat = pe_cat.reshape(2 * E, 128)
    qe_cat = qe_cat.reshape(2 * E, 128)

    pv_cat, pu_cat, qv_cat = _node_proj(
        h_v, h_u, W1_a2u[:D], W1_a2u[D:2 * D], W1_u2a[D:2 * D])
    pv_cat = pv_cat.reshape(2 * NV, 128)
    pu_cat = pu_cat.reshape(2 * NU, 128)
    qv_cat = qv_cat.reshape(2 * NV, 128)

    npad = -(-max(NU, NV) // (NS * 8)) * (NS * 8)
    zeros_rows = jnp.zeros((npad, 128), f32)

    cnt_a, cnt_b = _make_count_kernel(max(NU, NV), E)(idx_v, idx_u)
    cnt_u = cnt_b[:NU]
    cnt_v = cnt_a[:NV]

    (au_cat,) = _make_edge_pass(NV, NU, NU, E)(
        pv_cat, idx_v, pu_cat, idx_u, pe_cat, zeros_rows)

    h_u_out, qu_cat = _mid(
        au_cat[0, :NU], au_cat[1, :NU], cnt_u, h_u, S.reshape(1, 1),
        W2_a2u[:D], W2_a2u[D:], b2_a2u[None, :],
        W1_u[:D], W1_u[D:], b1_u[None, :],
        W2_u, b2_u[None, :],
        W1_u2a[:D])
    qu_cat = qu_cat.reshape(2 * NU, 128)

    (av_cat,) = _make_edge_pass(NU, NV, NV, E)(
        qu_cat, idx_u, qv_cat, idx_v, qe_cat, zeros_rows)

    h_v_out = _final(
        av_cat[0, :NV], av_cat[1, :NV], cnt_v, h_v,
        W2_u2a[:D], W2_u2a[D:], b2_u2a[None, :],
        W1_a[:D], W1_a[D:], b1_a[None, :],
        W2_a, b2_a[None, :])

    return (h_v_out, h_u_out)
